# Initial kernel scaffold; baseline (speedup 1.0000x reference)
import jax
import jax.numpy as jnp
from jax.experimental import pallas as pl


def kernel(vis_embed, sem_embed, W1, b1, W2, b2, Wg, bg):
    raise NotImplementedError("write your pallas kernel here")


# speedup vs baseline: 2.4266x; 2.4266x over previous
import functools

import jax
import jax.numpy as jnp
from jax.experimental import pallas as pl
from jax.experimental.pallas import tpu as pltpu

BB = 16


def _fused_kernel(vis_ref, sem_ref, w12_ref, b12_ref, wg_ref, bg_ref, out_ref,
                  *, n_img, n_know, hid, hid_adj):
    n = n_img + n_know
    x = jnp.concatenate([vis_ref[...], sem_ref[...]], axis=1)
    xb = x.astype(jnp.bfloat16)
    x2d = xb.reshape(BB * n, hid)

    ab = jax.lax.dot_general(
        x2d, w12_ref[...], (((1,), (0,)), ((), ())),
        preferred_element_type=jnp.float32) + b12_ref[...]
    a = ab[:, :hid_adj].astype(jnp.bfloat16).reshape(BB, n, hid_adj)
    b = ab[:, hid_adj:].astype(jnp.bfloat16).reshape(BB, n, hid_adj)

    aggs = []
    inv_s = []
    for i in range(BB):
        logits = jax.lax.dot_general(
            a[i], b[i], (((1,), (1,)), ((), ())),
            preferred_element_type=jnp.float32)
        m = jnp.max(logits, axis=-1, keepdims=True)
        e = jnp.exp(logits - m)
        s = jnp.sum(e, axis=-1, keepdims=True)
        agg = jax.lax.dot_general(
            e.astype(jnp.bfloat16), xb[i], (((1,), (0,)), ((), ())),
            preferred_element_type=jnp.float32)
        aggs.append(agg)
        inv_s.append(1.0 / s)
    agg_all = jnp.concatenate(aggs, axis=0).astype(jnp.bfloat16)
    inv_s_all = jnp.concatenate(inv_s, axis=0)

    hw = jax.lax.dot_general(
        agg_all, wg_ref[...], (((1,), (0,)), ((), ())),
        preferred_element_type=jnp.float32)
    h = jnp.maximum(hw * inv_s_all + bg_ref[...], 0.0)

    out_ref[...] = jnp.mean(h.reshape(BB, n, hid), axis=1)


def kernel(vis_embed, sem_embed, W1, b1, W2, b2, Wg, bg):
    bs, n_img, hid = vis_embed.shape
    n_know = sem_embed.shape[1]
    hid_adj = W1.shape[1]

    w12 = jnp.concatenate([W1, W2], axis=1).astype(jnp.bfloat16)
    b12 = jnp.concatenate([b1, b2]).reshape(1, 2 * hid_adj)
    wg = Wg.astype(jnp.bfloat16)
    bg2 = bg.reshape(1, hid)

    grid = bs // BB
    body = functools.partial(
        _fused_kernel, n_img=n_img, n_know=n_know, hid=hid, hid_adj=hid_adj)
    return pl.pallas_call(
        body,
        grid=(grid,),
        in_specs=[
            pl.BlockSpec((BB, n_img, hid), lambda i: (i, 0, 0)),
            pl.BlockSpec((BB, n_know, hid), lambda i: (i, 0, 0)),
            pl.BlockSpec((hid, 2 * hid_adj), lambda i: (0, 0)),
            pl.BlockSpec((1, 2 * hid_adj), lambda i: (0, 0)),
            pl.BlockSpec((hid, hid), lambda i: (0, 0)),
            pl.BlockSpec((1, hid), lambda i: (0, 0)),
        ],
        out_specs=pl.BlockSpec((BB, hid), lambda i: (i, 0)),
        out_shape=jax.ShapeDtypeStruct((bs, hid), jnp.float32),
        compiler_params=pltpu.CompilerParams(
            dimension_semantics=("arbitrary",)),
    )(vis_embed, sem_embed, w12, b12, wg, bg2)
